# baseline (speedup 1.0000x reference)
import functools

import jax
import jax.numpy as jnp
from jax.experimental import pallas as pl
from jax.experimental.pallas import tpu as pltpu
from jax.sharding import Mesh, PartitionSpec as P

K = 1024
D = 32
COMMITMENT_COST = 0.25


def _vq_kernel(z_ref, cb_ref, zq_ref, idx_ref, cts_ref, sq_ref):
    cb = cb_ref[...]
    BB = z_ref.shape[0]
    T = z_ref.shape[2]

    cn = jnp.sum(cb ** 2, axis=1, keepdims=True)
    cbm2 = cb * (-2.0)

    cts = jnp.zeros((K, 1), jnp.float32)
    sq = jnp.float32(0.0)
    for i in range(BB):
        zb = z_ref[i]
        xn = jnp.sum(zb ** 2, axis=0, keepdims=True)

        s2 = jax.lax.dot_general(cbm2, zb, (((1,), (0,)), ((), ())),
                                 preferred_element_type=jnp.float32)
        scores = (xn + s2) + cn

        m = jnp.min(scores, axis=0, keepdims=True)
        iota_k = jax.lax.broadcasted_iota(jnp.int32, (K, T), 0)
        masked = jnp.where(scores <= m, iota_k, K)
        idx = jnp.min(masked, axis=0, keepdims=True)
        idx_ref[i] = idx

        oh = (masked == idx).astype(jnp.float32)

        zq = jax.lax.dot_general(cb, oh, (((0,), (0,)), ((), ())),
                                 preferred_element_type=jnp.float32)
        zq_ref[i] = zq

        cts = cts + jnp.sum(oh, axis=1, keepdims=True)
        diff = zb - zq
        sq = sq + jnp.sum(diff * diff)

    cts_ref[...] = cts
    sq_ref[0, 0] = sq


def _finalize_kernel(n_rows, cts_ref, sq_ref, loss_ref, perp_ref):
    loss_ref[0, 0] = sq_ref[0, 0] * (COMMITMENT_COST / (n_rows * D))
    p = cts_ref[...] / jnp.float32(n_rows)
    perp_ref[0, 0] = jnp.exp(-jnp.sum(p * jnp.log(p + 1e-10)))


def _per_shard(z, codebook, n_rows):
    BB, Dd, T = z.shape
    zq, idx3, cts, sq = pl.pallas_call(
        _vq_kernel,
        grid=(1,),
        in_specs=[
            pl.BlockSpec((BB, Dd, T), lambda b: (0, 0, 0)),
            pl.BlockSpec((K, Dd), lambda b: (0, 0)),
        ],
        out_specs=[
            pl.BlockSpec((BB, Dd, T), lambda b: (0, 0, 0)),
            pl.BlockSpec((BB, 1, T), lambda b: (0, 0, 0)),
            pl.BlockSpec((K, 1), lambda b: (0, 0)),
            pl.BlockSpec(memory_space=pltpu.SMEM),
        ],
        out_shape=[
            jax.ShapeDtypeStruct((BB, Dd, T), jnp.float32),
            jax.ShapeDtypeStruct((BB, 1, T), jnp.int32),
            jax.ShapeDtypeStruct((K, 1), jnp.float32),
            jax.ShapeDtypeStruct((1, 1), jnp.float32),
        ],
    )(z, codebook)

    cts = jax.lax.psum(cts, "x")
    sq = jax.lax.psum(sq, "x")

    loss, perp = pl.pallas_call(
        functools.partial(_finalize_kernel, n_rows),
        in_specs=[
            pl.BlockSpec((K, 1), lambda: (0, 0)),
            pl.BlockSpec(memory_space=pltpu.SMEM),
        ],
        out_specs=[
            pl.BlockSpec(memory_space=pltpu.SMEM),
            pl.BlockSpec(memory_space=pltpu.SMEM),
        ],
        out_shape=[
            jax.ShapeDtypeStruct((1, 1), jnp.float32),
            jax.ShapeDtypeStruct((1, 1), jnp.float32),
        ],
    )(cts, sq)
    return zq, idx3, loss, perp


@jax.jit
def kernel(z, codebook):
    B, Dd, T = z.shape
    devs = jax.devices()
    ndev = 2 if len(devs) >= 2 and B % 2 == 0 else 1
    mesh = Mesh(devs[:ndev], ("x",))
    fn = jax.shard_map(
        functools.partial(_per_shard, n_rows=B * T),
        mesh=mesh,
        in_specs=(P("x"), P()),
        out_specs=(P("x"), P("x"), P(), P()),
        check_vma=False,
    )
    zq, idx3, loss, perp = fn(z, codebook)
    return (zq, loss[0, 0], perp[0, 0], idx3.reshape(B, T))


# speedup vs baseline: 11.7007x; 11.7007x over previous
import jax
import jax.numpy as jnp
from jax.experimental import pallas as pl
from jax.experimental.pallas import tpu as pltpu

K = 1024
D = 32
COMMITMENT_COST = 0.25


def _vq_kernel(z_ref, cb_ref, zq_ref, idx_ref, loss_ref, perp_ref, counts_scr):
    b = pl.program_id(0)
    nb = pl.num_programs(0)

    cb = cb_ref[...]
    BB = z_ref.shape[0]
    T = z_ref.shape[2]

    cn = jnp.sum(cb ** 2, axis=1, keepdims=True)
    cbm2 = cb * (-2.0)

    ones_t = jnp.ones((T, 8), jnp.float32)
    cts = jnp.zeros((K, 8), jnp.float32)
    sq = jnp.float32(0.0)
    for i in range(BB):
        zb = z_ref[i]
        xn = jnp.sum(zb ** 2, axis=0, keepdims=True)

        s2 = jax.lax.dot_general(cbm2, zb, (((1,), (0,)), ((), ())),
                                 preferred_element_type=jnp.float32)
        scores = (xn + s2) + cn

        m = jnp.min(scores, axis=0, keepdims=True)
        iota_k = jax.lax.broadcasted_iota(jnp.int32, (K, T), 0)
        masked = jnp.where(scores <= m, iota_k, K)
        idx = jnp.min(masked, axis=0, keepdims=True)
        idx_ref[i] = idx

        oh = (masked == idx).astype(jnp.float32)

        zq = jax.lax.dot_general(cb, oh, (((0,), (0,)), ((), ())),
                                 preferred_element_type=jnp.float32)
        zq_ref[i] = zq

        cts = cts + jax.lax.dot_general(oh, ones_t, (((1,), (0,)), ((), ())),
                                        preferred_element_type=jnp.float32)
        diff = zb - zq
        sq = sq + jnp.sum(diff * diff)

    @pl.when(b == 0)
    def _init():
        counts_scr[...] = cts[:, 0:1]
        loss_ref[0, 0] = sq

    @pl.when(b != 0)
    def _acc():
        counts_scr[...] += cts[:, 0:1]
        loss_ref[0, 0] += sq

    @pl.when(b == nb - 1)
    def _finalize():
        n_rows = jnp.float32(nb * BB * T)
        loss_ref[0, 0] = loss_ref[0, 0] * (COMMITMENT_COST / (nb * BB * T * D))
        p = counts_scr[...] / n_rows
        perp_ref[0, 0] = jnp.exp(-jnp.sum(p * jnp.log(p + 1e-10)))


BATCH_BLOCK = 16


@jax.jit
def kernel(z, codebook):
    B, Dd, T = z.shape
    BB = BATCH_BLOCK
    zq, idx3, loss, perp = pl.pallas_call(
        _vq_kernel,
        grid=(B // BB,),
        in_specs=[
            pl.BlockSpec((BB, Dd, T), lambda b: (b, 0, 0)),
            pl.BlockSpec((K, Dd), lambda b: (0, 0)),
        ],
        out_specs=[
            pl.BlockSpec((BB, Dd, T), lambda b: (b, 0, 0)),
            pl.BlockSpec((BB, 1, T), lambda b: (b, 0, 0)),
            pl.BlockSpec(memory_space=pltpu.SMEM),
            pl.BlockSpec(memory_space=pltpu.SMEM),
        ],
        out_shape=[
            jax.ShapeDtypeStruct((B, Dd, T), jnp.float32),
            jax.ShapeDtypeStruct((B, 1, T), jnp.int32),
            jax.ShapeDtypeStruct((1, 1), jnp.float32),
            jax.ShapeDtypeStruct((1, 1), jnp.float32),
        ],
        scratch_shapes=[pltpu.VMEM((K, 1), jnp.float32)],
    )(z, codebook)
    return (zq, loss[0, 0], perp[0, 0], idx3.reshape(B, T))


# speedup vs baseline: 11.8914x; 1.0163x over previous
import jax
import jax.numpy as jnp
from jax.experimental import pallas as pl
from jax.experimental.pallas import tpu as pltpu

K = 1024
D = 32
COMMITMENT_COST = 0.25


def _vq_kernel(z_ref, cb_ref, zq_ref, idx_ref, loss_ref, perp_ref, counts_scr):
    b = pl.program_id(0)
    nb = pl.num_programs(0)

    cb = cb_ref[...]
    BB = z_ref.shape[0]
    T = z_ref.shape[2]

    cn = jnp.sum(cb ** 2, axis=1, keepdims=True)
    cbm2 = cb * (-2.0)

    cts = jnp.zeros((K, 1), jnp.float32)
    sq = jnp.float32(0.0)
    for i in range(BB):
        zb = z_ref[i]
        xn = jnp.sum(zb ** 2, axis=0, keepdims=True)

        s2 = jax.lax.dot_general(cbm2, zb, (((1,), (0,)), ((), ())),
                                 preferred_element_type=jnp.float32)
        scores = (xn + s2) + cn

        m = jnp.min(scores, axis=0, keepdims=True)
        iota_k = jax.lax.broadcasted_iota(jnp.int32, (K, T), 0)
        masked = jnp.where(scores <= m, iota_k, K)
        idx = jnp.min(masked, axis=0, keepdims=True)
        idx_ref[i] = idx

        oh = (masked == idx).astype(jnp.float32)

        zq = jax.lax.dot_general(cb, oh, (((0,), (0,)), ((), ())),
                                 preferred_element_type=jnp.float32)
        zq_ref[i] = zq

        cts = cts + jnp.sum(oh, axis=1, keepdims=True)
        diff = zb - zq
        sq = sq + jnp.sum(diff * diff)

    @pl.when(b == 0)
    def _init():
        counts_scr[...] = cts
        loss_ref[0, 0] = sq

    @pl.when(b != 0)
    def _acc():
        counts_scr[...] += cts
        loss_ref[0, 0] += sq

    @pl.when(b == nb - 1)
    def _finalize():
        n_rows = jnp.float32(nb * BB * T)
        loss_ref[0, 0] = loss_ref[0, 0] * (COMMITMENT_COST / (nb * BB * T * D))
        p = counts_scr[...] / n_rows
        perp_ref[0, 0] = jnp.exp(-jnp.sum(p * jnp.log(p + 1e-10)))


BATCH_BLOCK = 8


@jax.jit
def kernel(z, codebook):
    B, Dd, T = z.shape
    BB = BATCH_BLOCK
    zq, idx3, loss, perp = pl.pallas_call(
        _vq_kernel,
        grid=(B // BB,),
        in_specs=[
            pl.BlockSpec((BB, Dd, T), lambda b: (b, 0, 0)),
            pl.BlockSpec((K, Dd), lambda b: (0, 0)),
        ],
        out_specs=[
            pl.BlockSpec((BB, Dd, T), lambda b: (b, 0, 0)),
            pl.BlockSpec((BB, 1, T), lambda b: (b, 0, 0)),
            pl.BlockSpec(memory_space=pltpu.SMEM),
            pl.BlockSpec(memory_space=pltpu.SMEM),
        ],
        out_shape=[
            jax.ShapeDtypeStruct((B, Dd, T), jnp.float32),
            jax.ShapeDtypeStruct((B, 1, T), jnp.int32),
            jax.ShapeDtypeStruct((1, 1), jnp.float32),
            jax.ShapeDtypeStruct((1, 1), jnp.float32),
        ],
        scratch_shapes=[pltpu.VMEM((K, 1), jnp.float32)],
    )(z, codebook)
    return (zq, loss[0, 0], perp[0, 0], idx3.reshape(B, T))
